# baseline (speedup 1.0000x reference)
import functools

import jax
import jax.numpy as jnp
from jax import lax
from jax.experimental import pallas as pl
from jax.experimental.pallas import tpu as pltpu
from jax.experimental.pallas import tpu_sc as plsc

_SPLITS = 4
_SYMBOLS = 1024
_SUBDIM = 64

_NUM_CORES = 2
_NUM_SUBCORES = 16
_NW = _NUM_CORES * _NUM_SUBCORES

_CHUNK = 128


def _pq_tc_body(x_ref, w2_ref, cbsq_ref, idx_ref, err_ref, *, n_tokens):
    i = pl.program_id(0)
    err_local = jnp.float32(0.0)
    cols = []
    for s in range(_SPLITS):
        xi = x_ref[:, s * _SUBDIM:(s + 1) * _SUBDIM]
        w2 = w2_ref[s]
        cbsq = cbsq_ref[s]
        xsq = jnp.sum(xi * xi, axis=1, keepdims=True)
        prod2 = jax.lax.dot_general(
            xi, w2, (((1,), (1,)), ((), ())),
            preferred_element_type=jnp.float32)
        sc = (xsq + cbsq) - prod2
        m = jnp.min(sc, axis=1, keepdims=True)
        iota = jax.lax.broadcasted_iota(jnp.int32, sc.shape, 1
                                        ).astype(jnp.float32)
        idxf = jnp.min(jnp.where(sc == m, iota, jnp.float32(_SYMBOLS)),
                       axis=1, keepdims=True)
        cols.append(idxf.astype(jnp.int32) + (s * _SYMBOLS))
        err_local = err_local + jnp.sum(m)

    idx_ref[...] = jnp.concatenate(cols, axis=1)

    scale = jnp.float32(1.25 / (n_tokens * _SUBDIM))
    contrib = jnp.full((1, 1), err_local * scale, dtype=jnp.float32)

    @pl.when(i == 0)
    def _():
        err_ref[...] = contrib

    @pl.when(i > 0)
    def _():
        err_ref[...] = err_ref[...] + contrib


def _sc_gather_body(table_hbm, idx_hbm, out_hbm, idx_v, rows_v, sem):
    wid = lax.axis_index("s") * _NUM_CORES + lax.axis_index("c")
    bpw = idx_v.shape[0]
    base = wid * bpw
    pltpu.sync_copy(idx_hbm.at[pl.ds(base, bpw)], idx_v)
    pltpu.async_copy(table_hbm.at[idx_v], rows_v, sem).wait()
    pltpu.sync_copy(rows_v, out_hbm.at[pl.ds(base, bpw)])


@jax.jit
def kernel(x, W):
    B, T, F = x.shape
    n_tokens = B * T
    block_tokens = 2304
    n_halves = 2
    half_tokens = n_tokens // n_halves
    n_blocks = half_tokens // block_tokens
    x2 = x.reshape(n_tokens, F)

    W2 = W + W
    cbsq = jnp.sum(W * W, axis=-1)[:, None, :]
    Wf = W.reshape(_SPLITS * _SYMBOLS, _SUBDIM)

    tc = pl.pallas_call(
        functools.partial(_pq_tc_body, n_tokens=n_tokens),
        grid=(n_blocks,),
        in_specs=[
            pl.BlockSpec((block_tokens, F), lambda i: (i, 0)),
            pl.BlockSpec((_SPLITS, _SYMBOLS, _SUBDIM), lambda i: (0, 0, 0)),
            pl.BlockSpec((_SPLITS, 1, _SYMBOLS), lambda i: (0, 0, 0)),
        ],
        out_specs=[
            pl.BlockSpec((block_tokens, _SPLITS), lambda i: (i, 0)),
            pl.BlockSpec((1, 1), lambda i: (0, 0)),
        ],
        out_shape=[
            jax.ShapeDtypeStruct((half_tokens, _SPLITS), jnp.int32),
            jax.ShapeDtypeStruct((1, 1), jnp.float32),
        ],
    )

    n_rows = half_tokens * _SPLITS
    bpw = n_rows // _NW

    gather = functools.partial(
        pl.kernel,
        out_type=jax.ShapeDtypeStruct((n_rows, _SUBDIM), jnp.float32),
        mesh=plsc.VectorSubcoreMesh(core_axis_name="c", subcore_axis_name="s"),
        scratch_types=[
            pltpu.VMEM((bpw,), jnp.int32),
            pltpu.VMEM((bpw, _SUBDIM), jnp.float32),
            pltpu.SemaphoreType.DMA,
        ],
        compiler_params=pltpu.CompilerParams(use_tc_tiling_on_sc=False),
    )(_sc_gather_body)

    idxs = []
    errs = []
    for h in range(n_halves):
        idx_h, err_h = tc(x2[h * half_tokens:(h + 1) * half_tokens],
                          W2, cbsq)
        idxs.append(idx_h)
        errs.append(err_h)

    rows = [gather(Wf, idx_h.reshape(n_rows)) for idx_h in idxs]

    quant = jnp.concatenate(
        [r.reshape(B // n_halves, T, F) for r in rows], axis=0)
    err = errs[0][0, 0]
    for e in errs[1:]:
        err = err + e[0, 0]
    return quant, err


# speedup vs baseline: 1.1743x; 1.1743x over previous
import functools

import jax
import jax.numpy as jnp
from jax import lax
from jax.experimental import pallas as pl
from jax.experimental.pallas import tpu as pltpu
from jax.experimental.pallas import tpu_sc as plsc

_SPLITS = 4
_SYMBOLS = 1024
_SUBDIM = 64

_NUM_CORES = 2
_NUM_SUBCORES = 16
_NW = _NUM_CORES * _NUM_SUBCORES

_CHUNK = 128


def _pq_tc_body(x_ref, w2_ref, cbsq_ref, idx_ref, err_ref, *, n_tokens):
    i = pl.program_id(0)
    err_local = jnp.float32(0.0)
    cols = []
    for s in range(_SPLITS):
        xi = x_ref[:, s * _SUBDIM:(s + 1) * _SUBDIM]
        w2 = w2_ref[s]
        cbsq = cbsq_ref[s]
        xsq = jnp.sum(xi * xi, axis=1, keepdims=True)
        prod2 = jax.lax.dot_general(
            xi, w2, (((1,), (1,)), ((), ())),
            preferred_element_type=jnp.float32)
        sc = (xsq + cbsq) - prod2
        m = jnp.min(sc, axis=1, keepdims=True)
        iota = jax.lax.broadcasted_iota(jnp.int32, sc.shape, 1
                                        ).astype(jnp.float32)
        idxf = jnp.min(jnp.where(sc == m, iota, jnp.float32(_SYMBOLS)),
                       axis=1, keepdims=True)
        cols.append(idxf.astype(jnp.int32) + (s * _SYMBOLS))
        err_local = err_local + jnp.sum(m)

    idx_ref[...] = jnp.concatenate(cols, axis=1)

    scale = jnp.float32(1.25 / (n_tokens * _SUBDIM))
    contrib = jnp.full((1, 1), err_local * scale, dtype=jnp.float32)

    @pl.when(i == 0)
    def _():
        err_ref[...] = contrib

    @pl.when(i > 0)
    def _():
        err_ref[...] = err_ref[...] + contrib


def _sc_gather_body(table_hbm, idx_hbm, out_hbm, idx_v, rows_v, sem):
    wid = lax.axis_index("s") * _NUM_CORES + lax.axis_index("c")
    bpw = idx_v.shape[0]
    base = wid * bpw
    pltpu.sync_copy(idx_hbm.at[pl.ds(base, bpw)], idx_v)
    pltpu.async_copy(table_hbm.at[idx_v], rows_v, sem).wait()
    pltpu.sync_copy(rows_v, out_hbm.at[pl.ds(base, bpw)])


@jax.jit
def kernel(x, W):
    B, T, F = x.shape
    n_tokens = B * T
    block_tokens = 2304
    n_halves = 1
    half_tokens = n_tokens // n_halves
    n_blocks = half_tokens // block_tokens
    x2 = x.reshape(n_tokens, F)

    W2 = W + W
    cbsq = jnp.sum(W * W, axis=-1)[:, None, :]
    Wf = W.reshape(_SPLITS * _SYMBOLS, _SUBDIM)

    tc = pl.pallas_call(
        functools.partial(_pq_tc_body, n_tokens=n_tokens),
        grid=(n_blocks,),
        in_specs=[
            pl.BlockSpec((block_tokens, F), lambda i: (i, 0)),
            pl.BlockSpec((_SPLITS, _SYMBOLS, _SUBDIM), lambda i: (0, 0, 0)),
            pl.BlockSpec((_SPLITS, 1, _SYMBOLS), lambda i: (0, 0, 0)),
        ],
        out_specs=[
            pl.BlockSpec((block_tokens, _SPLITS), lambda i: (i, 0)),
            pl.BlockSpec((1, 1), lambda i: (0, 0)),
        ],
        out_shape=[
            jax.ShapeDtypeStruct((half_tokens, _SPLITS), jnp.int32),
            jax.ShapeDtypeStruct((1, 1), jnp.float32),
        ],
    )

    n_rows = half_tokens * _SPLITS
    bpw = n_rows // _NW

    gather = functools.partial(
        pl.kernel,
        out_type=jax.ShapeDtypeStruct((n_rows, _SUBDIM), jnp.float32),
        mesh=plsc.VectorSubcoreMesh(core_axis_name="c", subcore_axis_name="s"),
        scratch_types=[
            pltpu.VMEM((bpw,), jnp.int32),
            pltpu.VMEM((bpw, _SUBDIM), jnp.float32),
            pltpu.SemaphoreType.DMA,
        ],
        compiler_params=pltpu.CompilerParams(use_tc_tiling_on_sc=False),
    )(_sc_gather_body)

    idxs = []
    errs = []
    for h in range(n_halves):
        idx_h, err_h = tc(x2[h * half_tokens:(h + 1) * half_tokens],
                          W2, cbsq)
        idxs.append(idx_h)
        errs.append(err_h)

    rows = [gather(Wf, idx_h.reshape(n_rows)) for idx_h in idxs]

    quant = jnp.concatenate(
        [r.reshape(B // n_halves, T, F) for r in rows], axis=0)
    err = errs[0][0, 0]
    for e in errs[1:]:
        err = err + e[0, 0]
    return quant, err
